# baseline (speedup 1.0000x reference)
import functools

import jax
import jax.numpy as jnp
from jax import lax
from jax.experimental import pallas as pl
from jax.experimental.pallas import tpu as pltpu
from jax.experimental.pallas import tpu_sc as plsc

EMBED = 128
CH = 128


NBUF = 4


def _sc_gather3(li2, ci2, ui2, lang_tab, cons_tab, univ_tab, part, n_parts):
    B = li2.size // n_parts
    info = plsc.get_sparse_core_info()
    nc, ns = info.num_cores, info.num_subcores
    nw = nc * ns
    b_per_w = B // nw
    n_ch = b_per_w // CH
    part_row0 = part * (B // CH)
    mesh = plsc.VectorSubcoreMesh(core_axis_name="c", subcore_axis_name="s")

    @functools.partial(
        pl.kernel,
        mesh=mesh,
        out_type=(
            jax.ShapeDtypeStruct((B, EMBED), jnp.float32),
            jax.ShapeDtypeStruct((B, EMBED), jnp.float32),
            jax.ShapeDtypeStruct((B, EMBED), jnp.float32),
        ),
        scratch_types=[
            pltpu.VMEM((n_ch, CH), jnp.int32),
            pltpu.VMEM((n_ch, CH), jnp.int32),
            pltpu.VMEM((n_ch, CH), jnp.int32),
            pltpu.VMEM((NBUF, CH, EMBED), jnp.float32),
            pltpu.SemaphoreType.DMA((NBUF,)),
            pltpu.SemaphoreType.DMA((NBUF,)),
            pltpu.SemaphoreType.DMA,
        ],
    )
    def k(li, ci, ui, lt, ct, ut, ol, oc, ou, ixl, ixc, ixu, rows,
          sem_g, sem_w, sem_i):
        wid = lax.axis_index("s") * nc + lax.axis_index("c")
        base = wid * b_per_w
        row0 = part_row0 + wid * n_ch
        for src, dst in ((li, ixl), (ci, ixc), (ui, ixu)):
            pltpu.async_copy(src.at[pl.ds(row0, n_ch)], dst, sem_i).wait()

        tasks = []
        for ix, tab, out in ((ixl, lt, ol), (ixc, ct, oc), (ixu, ut, ou)):
            for j in range(n_ch):
                tasks.append((ix.at[j], tab, out.at[pl.ds(base + j * CH, CH)]))
        nt = len(tasks)

        gh = [None] * nt
        wh = [None] * nt

        def start_gather(t):
            ix, tab, _ = tasks[t]
            gh[t] = pltpu.async_copy(tab.at[ix], rows.at[t % NBUF],
                                     sem_g.at[t % NBUF])

        for t in range(min(NBUF, nt)):
            start_gather(t)
        for t in range(nt):
            gh[t].wait()
            _, _, out_slc = tasks[t]
            wh[t] = pltpu.async_copy(rows.at[t % NBUF], out_slc,
                                     sem_w.at[t % NBUF])
            nxt = t + NBUF
            if nxt < nt:
                wh[t].wait()
                start_gather(nxt)
        for t in range(max(0, nt - NBUF), nt):
            wh[t].wait()

    return k(li2, ci2, ui2, lang_tab, cons_tab, univ_tab)


N_PARTS = 2


def _dense_body(le, ce, ue, fw, fb, g, bt, ow, ob, o):
    dn = (((1,), (1,)), ((), ()))
    x = lax.dot_general(le[...], fw[:, 0 * EMBED:1 * EMBED], dn,
                        preferred_element_type=jnp.float32)
    x += lax.dot_general(ce[...], fw[:, 1 * EMBED:2 * EMBED], dn,
                         preferred_element_type=jnp.float32)
    x += lax.dot_general(ue[...], fw[:, 2 * EMBED:3 * EMBED], dn,
                         preferred_element_type=jnp.float32)
    x += fb[...]
    mean = jnp.mean(x, axis=1, keepdims=True)
    xc = x - mean
    var = jnp.mean(xc * xc, axis=1, keepdims=True)
    xn = xc * lax.rsqrt(var + 1e-5) * g[...] + bt[...]
    o[...] = lax.dot_general(xn, ow[...], dn,
                             preferred_element_type=jnp.float32) + ob[...]


def _tc_dense(lang_e, cons_e, univ_e, fusion_W, fusion_b, ln_gamma, ln_beta,
              out_W, out_b):
    B = lang_e.shape[0]
    blk = 2048
    grid = (B // blk,)
    emb_spec = pl.BlockSpec((blk, EMBED), lambda i: (i, 0))
    full = lambda r, c: pl.BlockSpec((r, c), lambda i: (0, 0))
    return pl.pallas_call(
        _dense_body,
        grid=grid,
        in_specs=[
            emb_spec, emb_spec, emb_spec,
            full(EMBED, 3 * EMBED),
            full(1, EMBED), full(1, EMBED), full(1, EMBED),
            full(EMBED, EMBED), full(1, EMBED),
        ],
        out_specs=emb_spec,
        out_shape=jax.ShapeDtypeStruct((B, EMBED), jnp.float32),
    )(lang_e, cons_e, univ_e, fusion_W,
      fusion_b.reshape(1, EMBED), ln_gamma.reshape(1, EMBED),
      ln_beta.reshape(1, EMBED), out_W, out_b.reshape(1, EMBED))


def kernel(language_input, consciousness_input, universe_input, lang_table,
           cons_table, univ_table, fusion_W, fusion_b, ln_gamma, ln_beta,
           out_W, out_b):
    B = language_input.shape[0]
    li2 = language_input.astype(jnp.int32).reshape(B // CH, CH)
    ci2 = consciousness_input.astype(jnp.int32).reshape(B // CH, CH)
    ui2 = universe_input.astype(jnp.int32).reshape(B // CH, CH)
    outs = []
    for p in range(N_PARTS):
        le, ce, ue = _sc_gather3(li2, ci2, ui2,
                                 lang_table, cons_table, univ_table,
                                 p, N_PARTS)
        outs.append(_tc_dense(le, ce, ue, fusion_W, fusion_b, ln_gamma,
                              ln_beta, out_W, out_b))
    return jnp.concatenate(outs, axis=0)


# speedup vs baseline: 1.1197x; 1.1197x over previous
import functools

import jax
import jax.numpy as jnp
from jax import lax
from jax.experimental import pallas as pl
from jax.experimental.pallas import tpu as pltpu
from jax.experimental.pallas import tpu_sc as plsc

EMBED = 128
CH = 128


NBUF = 6


def _sc_gather3(li2, ci2, ui2, lang_tab, cons_tab, univ_tab, part, n_parts):
    B = li2.size // n_parts
    info = plsc.get_sparse_core_info()
    nc, ns = info.num_cores, info.num_subcores
    nw = nc * ns
    b_per_w = B // nw
    n_ch = b_per_w // CH
    part_row0 = part * (B // CH)
    mesh = plsc.VectorSubcoreMesh(core_axis_name="c", subcore_axis_name="s")

    @functools.partial(
        pl.kernel,
        mesh=mesh,
        out_type=(
            jax.ShapeDtypeStruct((B, EMBED), jnp.float32),
            jax.ShapeDtypeStruct((B, EMBED), jnp.float32),
            jax.ShapeDtypeStruct((B, EMBED), jnp.float32),
        ),
        scratch_types=[
            pltpu.VMEM((n_ch, CH), jnp.int32),
            pltpu.VMEM((n_ch, CH), jnp.int32),
            pltpu.VMEM((n_ch, CH), jnp.int32),
            pltpu.VMEM((NBUF, CH, EMBED), jnp.float32),
            pltpu.SemaphoreType.DMA((NBUF,)),
            pltpu.SemaphoreType.DMA((NBUF,)),
            pltpu.SemaphoreType.DMA,
        ],
    )
    def k(li, ci, ui, lt, ct, ut, ol, oc, ou, ixl, ixc, ixu, rows,
          sem_g, sem_w, sem_i):
        wid = lax.axis_index("s") * nc + lax.axis_index("c")
        base = wid * b_per_w
        row0 = part_row0 + wid * n_ch
        for src, dst in ((li, ixl), (ci, ixc), (ui, ixu)):
            pltpu.async_copy(src.at[pl.ds(row0, n_ch)], dst, sem_i).wait()

        tasks = []
        for ix, tab, out in ((ixl, lt, ol), (ixc, ct, oc), (ixu, ut, ou)):
            for j in range(n_ch):
                tasks.append((ix.at[j], tab, out.at[pl.ds(base + j * CH, CH)]))
        nt = len(tasks)

        gh = [None] * nt
        wh = [None] * nt

        def start_gather(t):
            ix, tab, _ = tasks[t]
            gh[t] = pltpu.async_copy(tab.at[ix], rows.at[t % NBUF],
                                     sem_g.at[t % NBUF])

        for t in range(min(NBUF, nt)):
            start_gather(t)
        for t in range(nt):
            gh[t].wait()
            _, _, out_slc = tasks[t]
            wh[t] = pltpu.async_copy(rows.at[t % NBUF], out_slc,
                                     sem_w.at[t % NBUF])
            nxt = t + NBUF
            if nxt < nt:
                wh[t].wait()
                start_gather(nxt)
        for t in range(max(0, nt - NBUF), nt):
            wh[t].wait()

    return k(li2, ci2, ui2, lang_tab, cons_tab, univ_tab)


N_PARTS = 1


def _dense_body(le, ce, ue, fw, fb, g, bt, ow, ob, o):
    dn = (((1,), (1,)), ((), ()))
    x = lax.dot_general(le[...], fw[:, 0 * EMBED:1 * EMBED], dn,
                        preferred_element_type=jnp.float32)
    x += lax.dot_general(ce[...], fw[:, 1 * EMBED:2 * EMBED], dn,
                         preferred_element_type=jnp.float32)
    x += lax.dot_general(ue[...], fw[:, 2 * EMBED:3 * EMBED], dn,
                         preferred_element_type=jnp.float32)
    x += fb[...]
    mean = jnp.mean(x, axis=1, keepdims=True)
    xc = x - mean
    var = jnp.mean(xc * xc, axis=1, keepdims=True)
    xn = xc * lax.rsqrt(var + 1e-5) * g[...] + bt[...]
    o[...] = lax.dot_general(xn, ow[...], dn,
                             preferred_element_type=jnp.float32) + ob[...]


def _tc_dense(lang_e, cons_e, univ_e, fusion_W, fusion_b, ln_gamma, ln_beta,
              out_W, out_b):
    B = lang_e.shape[0]
    blk = 2048
    grid = (B // blk,)
    emb_spec = pl.BlockSpec((blk, EMBED), lambda i: (i, 0))
    full = lambda r, c: pl.BlockSpec((r, c), lambda i: (0, 0))
    return pl.pallas_call(
        _dense_body,
        grid=grid,
        in_specs=[
            emb_spec, emb_spec, emb_spec,
            full(EMBED, 3 * EMBED),
            full(1, EMBED), full(1, EMBED), full(1, EMBED),
            full(EMBED, EMBED), full(1, EMBED),
        ],
        out_specs=emb_spec,
        out_shape=jax.ShapeDtypeStruct((B, EMBED), jnp.float32),
    )(lang_e, cons_e, univ_e, fusion_W,
      fusion_b.reshape(1, EMBED), ln_gamma.reshape(1, EMBED),
      ln_beta.reshape(1, EMBED), out_W, out_b.reshape(1, EMBED))


def kernel(language_input, consciousness_input, universe_input, lang_table,
           cons_table, univ_table, fusion_W, fusion_b, ln_gamma, ln_beta,
           out_W, out_b):
    B = language_input.shape[0]
    li2 = language_input.astype(jnp.int32).reshape(B // CH, CH)
    ci2 = consciousness_input.astype(jnp.int32).reshape(B // CH, CH)
    ui2 = universe_input.astype(jnp.int32).reshape(B // CH, CH)
    outs = []
    for p in range(N_PARTS):
        le, ce, ue = _sc_gather3(li2, ci2, ui2,
                                 lang_table, cons_table, univ_table,
                                 p, N_PARTS)
        outs.append(_tc_dense(le, ce, ue, fusion_W, fusion_b, ln_gamma,
                              ln_beta, out_W, out_b))
    return jnp.concatenate(outs, axis=0)


# speedup vs baseline: 1.1513x; 1.0282x over previous
import functools

import jax
import jax.numpy as jnp
from jax import lax
from jax.experimental import pallas as pl
from jax.experimental.pallas import tpu as pltpu
from jax.experimental.pallas import tpu_sc as plsc

EMBED = 128
CH = 128


NBUF = 6


def _sc_gather3(li2, ci2, ui2, lang_tab, cons_tab, univ_tab, part, n_parts):
    B = li2.size // n_parts
    info = plsc.get_sparse_core_info()
    nc, ns = info.num_cores, info.num_subcores
    nw = nc * ns
    b_per_w = B // nw
    n_ch = b_per_w // CH
    part_row0 = part * (B // CH)
    mesh = plsc.VectorSubcoreMesh(core_axis_name="c", subcore_axis_name="s")

    @functools.partial(
        pl.kernel,
        mesh=mesh,
        out_type=jax.ShapeDtypeStruct((B, 3 * EMBED), jnp.float32),
        scratch_types=[
            pltpu.VMEM((n_ch, CH), jnp.int32),
            pltpu.VMEM((n_ch, CH), jnp.int32),
            pltpu.VMEM((n_ch, CH), jnp.int32),
            pltpu.VMEM((NBUF, CH, EMBED), jnp.float32),
            pltpu.SemaphoreType.DMA((NBUF,)),
            pltpu.SemaphoreType.DMA((NBUF,)),
            pltpu.SemaphoreType.DMA,
        ],
    )
    def k(li, ci, ui, lt, ct, ut, ocat, ixl, ixc, ixu, rows,
          sem_g, sem_w, sem_i):
        wid = lax.axis_index("s") * nc + lax.axis_index("c")
        base = wid * b_per_w
        row0 = part_row0 + wid * n_ch
        for src, dst in ((li, ixl), (ci, ixc), (ui, ixu)):
            pltpu.async_copy(src.at[pl.ds(row0, n_ch)], dst, sem_i).wait()

        tasks = []
        for col, (ix, tab) in enumerate(((ixl, lt), (ixc, ct), (ixu, ut))):
            for j in range(n_ch):
                tasks.append((ix.at[j], tab,
                              ocat.at[pl.ds(base + j * CH, CH),
                                      pl.ds(col * EMBED, EMBED)]))
        nt = len(tasks)

        gh = [None] * nt
        wh = [None] * nt

        def start_gather(t):
            ix, tab, _ = tasks[t]
            gh[t] = pltpu.async_copy(tab.at[ix], rows.at[t % NBUF],
                                     sem_g.at[t % NBUF])

        for t in range(min(NBUF, nt)):
            start_gather(t)
        for t in range(nt):
            gh[t].wait()
            _, _, out_slc = tasks[t]
            wh[t] = pltpu.async_copy(rows.at[t % NBUF], out_slc,
                                     sem_w.at[t % NBUF])
            nxt = t + NBUF
            if nxt < nt:
                wh[t].wait()
                start_gather(nxt)
        for t in range(max(0, nt - NBUF), nt):
            wh[t].wait()

    return k(li2, ci2, ui2, lang_tab, cons_tab, univ_tab)


N_PARTS = 1


def _dense_body(fe, fw, fb, g, bt, ow, ob, o):
    dn = (((1,), (1,)), ((), ()))
    x = lax.dot_general(fe[...], fw[...], dn,
                        preferred_element_type=jnp.float32)
    x += fb[...]
    mean = jnp.mean(x, axis=1, keepdims=True)
    xc = x - mean
    var = jnp.mean(xc * xc, axis=1, keepdims=True)
    xn = xc * lax.rsqrt(var + 1e-5) * g[...] + bt[...]
    o[...] = lax.dot_general(xn, ow[...], dn,
                             preferred_element_type=jnp.float32) + ob[...]


def _tc_dense(fused_e, fusion_W, fusion_b, ln_gamma, ln_beta, out_W, out_b):
    B = fused_e.shape[0]
    blk = 2048
    grid = (B // blk,)
    full = lambda r, c: pl.BlockSpec((r, c), lambda i: (0, 0))
    return pl.pallas_call(
        _dense_body,
        grid=grid,
        in_specs=[
            pl.BlockSpec((blk, 3 * EMBED), lambda i: (i, 0)),
            full(EMBED, 3 * EMBED),
            full(1, EMBED), full(1, EMBED), full(1, EMBED),
            full(EMBED, EMBED), full(1, EMBED),
        ],
        out_specs=pl.BlockSpec((blk, EMBED), lambda i: (i, 0)),
        out_shape=jax.ShapeDtypeStruct((B, EMBED), jnp.float32),
    )(fused_e, fusion_W,
      fusion_b.reshape(1, EMBED), ln_gamma.reshape(1, EMBED),
      ln_beta.reshape(1, EMBED), out_W, out_b.reshape(1, EMBED))


def kernel(language_input, consciousness_input, universe_input, lang_table,
           cons_table, univ_table, fusion_W, fusion_b, ln_gamma, ln_beta,
           out_W, out_b):
    B = language_input.shape[0]
    li2 = language_input.astype(jnp.int32).reshape(B // CH, CH)
    ci2 = consciousness_input.astype(jnp.int32).reshape(B // CH, CH)
    ui2 = universe_input.astype(jnp.int32).reshape(B // CH, CH)
    outs = []
    for p in range(N_PARTS):
        fe = _sc_gather3(li2, ci2, ui2, lang_table, cons_table, univ_table,
                         p, N_PARTS)
        outs.append(_tc_dense(fe, fusion_W, fusion_b, ln_gamma,
                              ln_beta, out_W, out_b))
    return jnp.concatenate(outs, axis=0)


# speedup vs baseline: 1.1670x; 1.0136x over previous
import functools

import jax
import jax.numpy as jnp
from jax import lax
from jax.experimental import pallas as pl
from jax.experimental.pallas import tpu as pltpu
from jax.experimental.pallas import tpu_sc as plsc

EMBED = 128
CH = 128


NBUF = 6


def _sc_gather3(li2, ci2, ui2, lang_tab, cons_tab, univ_tab, part, n_parts):
    B = li2.size // n_parts
    info = plsc.get_sparse_core_info()
    nc, ns = info.num_cores, info.num_subcores
    nw = nc * ns
    b_per_w = B // nw
    n_ch = b_per_w // CH
    part_row0 = part * (B // CH)
    mesh = plsc.VectorSubcoreMesh(core_axis_name="c", subcore_axis_name="s")

    @functools.partial(
        pl.kernel,
        mesh=mesh,
        out_type=(
            jax.ShapeDtypeStruct((B, EMBED), jnp.float32),
            jax.ShapeDtypeStruct((B, EMBED), jnp.float32),
            jax.ShapeDtypeStruct((B, EMBED), jnp.float32),
        ),
        scratch_types=[
            pltpu.VMEM((n_ch, CH), jnp.int32),
            pltpu.VMEM((n_ch, CH), jnp.int32),
            pltpu.VMEM((n_ch, CH), jnp.int32),
            pltpu.VMEM((NBUF, CH, EMBED), jnp.float32),
            pltpu.SemaphoreType.DMA((NBUF,)),
            pltpu.SemaphoreType.DMA((NBUF,)),
            pltpu.SemaphoreType.DMA,
        ],
    )
    def k(li, ci, ui, lt, ct, ut, ol, oc, ou, ixl, ixc, ixu, rows,
          sem_g, sem_w, sem_i):
        wid = lax.axis_index("s") * nc + lax.axis_index("c")
        base = wid * b_per_w
        row0 = part_row0 + wid * n_ch
        for src, dst in ((li, ixl), (ci, ixc), (ui, ixu)):
            pltpu.async_copy(src.at[pl.ds(row0, n_ch)], dst, sem_i).wait()

        tasks = []
        for ix, tab, out in ((ixl, lt, ol), (ixc, ct, oc), (ixu, ut, ou)):
            for j in range(n_ch):
                tasks.append((ix.at[j], tab, out.at[pl.ds(base + j * CH, CH)]))
        nt = len(tasks)

        gh = [None] * nt
        wh = [None] * nt

        def start_gather(t):
            ix, tab, _ = tasks[t]
            gh[t] = pltpu.async_copy(tab.at[ix], rows.at[t % NBUF],
                                     sem_g.at[t % NBUF])

        for t in range(min(NBUF, nt)):
            start_gather(t)
        for t in range(nt):
            gh[t].wait()
            _, _, out_slc = tasks[t]
            wh[t] = pltpu.async_copy(rows.at[t % NBUF], out_slc,
                                     sem_w.at[t % NBUF])
            nxt = t + NBUF
            if nxt < nt:
                wh[t].wait()
                start_gather(nxt)
        for t in range(max(0, nt - NBUF), nt):
            wh[t].wait()

    return k(li2, ci2, ui2, lang_tab, cons_tab, univ_tab)


N_PARTS = 1


def _dense_body(le, ce, ue, fw, fb, g, bt, ow, ob, o):
    dn = (((1,), (1,)), ((), ()))
    x = lax.dot_general(le[...], fw[:, 0 * EMBED:1 * EMBED], dn,
                        preferred_element_type=jnp.float32)
    x += lax.dot_general(ce[...], fw[:, 1 * EMBED:2 * EMBED], dn,
                         preferred_element_type=jnp.float32)
    x += lax.dot_general(ue[...], fw[:, 2 * EMBED:3 * EMBED], dn,
                         preferred_element_type=jnp.float32)
    x += fb[...]
    mean = jnp.mean(x, axis=1, keepdims=True)
    xc = x - mean
    var = jnp.mean(xc * xc, axis=1, keepdims=True)
    xn = xc * lax.rsqrt(var + 1e-5) * g[...] + bt[...]
    o[...] = lax.dot_general(xn, ow[...], dn,
                             preferred_element_type=jnp.float32) + ob[...]


def _tc_dense(lang_e, cons_e, univ_e, fusion_W, fusion_b, ln_gamma, ln_beta,
              out_W, out_b):
    B = lang_e.shape[0]
    blk = 4096
    grid = (B // blk,)
    emb_spec = pl.BlockSpec((blk, EMBED), lambda i: (i, 0))
    full = lambda r, c: pl.BlockSpec((r, c), lambda i: (0, 0))
    return pl.pallas_call(
        _dense_body,
        grid=grid,
        in_specs=[
            emb_spec, emb_spec, emb_spec,
            full(EMBED, 3 * EMBED),
            full(1, EMBED), full(1, EMBED), full(1, EMBED),
            full(EMBED, EMBED), full(1, EMBED),
        ],
        out_specs=emb_spec,
        out_shape=jax.ShapeDtypeStruct((B, EMBED), jnp.float32),
    )(lang_e, cons_e, univ_e, fusion_W,
      fusion_b.reshape(1, EMBED), ln_gamma.reshape(1, EMBED),
      ln_beta.reshape(1, EMBED), out_W, out_b.reshape(1, EMBED))


def kernel(language_input, consciousness_input, universe_input, lang_table,
           cons_table, univ_table, fusion_W, fusion_b, ln_gamma, ln_beta,
           out_W, out_b):
    B = language_input.shape[0]
    li2 = language_input.astype(jnp.int32).reshape(B // CH, CH)
    ci2 = consciousness_input.astype(jnp.int32).reshape(B // CH, CH)
    ui2 = universe_input.astype(jnp.int32).reshape(B // CH, CH)
    outs = []
    for p in range(N_PARTS):
        le, ce, ue = _sc_gather3(li2, ci2, ui2,
                                 lang_table, cons_table, univ_table,
                                 p, N_PARTS)
        outs.append(_tc_dense(le, ce, ue, fusion_W, fusion_b, ln_gamma,
                              ln_beta, out_W, out_b))
    return jnp.concatenate(outs, axis=0)
